# baseline (speedup 1.0000x reference)
import functools

import jax
import jax.numpy as jnp
from jax import lax
from jax.experimental import pallas as pl
from jax.experimental.pallas import tpu as pltpu
from jax.experimental.pallas import tpu_sc as plsc

_B = 4
_S = 8192
_D = 1024
_N = _B * _S

_NC = 2
_NS = 16
_NW = _NC * _NS
_BPW = _N // _NW
_C = 32
_NCH = _BPW // _C

_mesh = plsc.VectorSubcoreMesh(core_axis_name="c", subcore_axis_name="s")


@functools.partial(
    pl.kernel,
    mesh=_mesh,
    out_type=jax.ShapeDtypeStruct((_N, _D), jnp.float32),
    scratch_types=[
        pltpu.VMEM((_NCH, _C), jnp.int32),
        pltpu.VMEM((3, _C, _D), jnp.float32),
        pltpu.SemaphoreType.DMA,
        pltpu.SemaphoreType.DMA,
        pltpu.SemaphoreType.DMA,
        pltpu.SemaphoreType.DMA,
        pltpu.SemaphoreType.DMA,
        pltpu.SemaphoreType.DMA,
    ],
)
def _gather_rows(idx_hbm, table_hbm, out_hbm, idx_v, rows_v,
                 g0, g1, g2, s0, s1, s2):
    wid = lax.axis_index("s") * _NC + lax.axis_index("c")
    base = wid * _BPW
    pltpu.sync_copy(idx_hbm.at[wid], idx_v)

    gsem = (g0, g1, g2)
    ssem = (s0, s1, s2)

    def _gather(ch, b):
        pltpu.async_copy(table_hbm.at[idx_v.at[ch]], rows_v.at[b], gsem[b])

    def _gwait(b):
        pltpu.make_async_copy(
            table_hbm.at[idx_v.at[0]], rows_v.at[b], gsem[b]
        ).wait()

    def _scatter(ch, b):
        pltpu.async_copy(
            rows_v.at[b], out_hbm.at[pl.ds(base + ch * _C, _C)], ssem[b]
        )

    def _swait(b):
        pltpu.make_async_copy(
            rows_v.at[b], out_hbm.at[pl.ds(base, _C)], ssem[b]
        ).wait()

    _gather(0, 0)
    _gather(1, 1)
    _gwait(0)
    _scatter(0, 0)
    _gather(2, 2)

    def _body(g, carry):
        for k in range(3):
            ch = g * 3 + 1 + k
            b = (1 + k) % 3
            _gwait(b)
            _scatter(ch, b)

            @pl.when(ch + 2 < _NCH)
            def _():
                _swait(k)
                _gather(ch + 2, k)

        return carry

    lax.fori_loop(0, (_NCH - 2) // 3, _body, 0)

    _gwait(1)
    _scatter(_NCH - 1, 1)
    _swait(2)
    _swait(0)
    _swait(1)


def kernel(inputs, embedding):
    idx = inputs.reshape(_NW, _NCH, _C).astype(jnp.int32)
    out = _gather_rows(idx, embedding)
    return out.reshape(inputs.shape + (_D,))


# speedup vs baseline: 1.0103x; 1.0103x over previous
import functools

import jax
import jax.numpy as jnp
from jax import lax
from jax.experimental import pallas as pl
from jax.experimental.pallas import tpu as pltpu
from jax.experimental.pallas import tpu_sc as plsc

_B = 4
_S = 8192
_D = 1024
_N = _B * _S

_NC = 2
_NS = 16
_NW = _NC * _NS
_BPW = _N // _NW
_C = 32
_NCH = _BPW // _C

_mesh = plsc.VectorSubcoreMesh(core_axis_name="c", subcore_axis_name="s")


@functools.partial(
    pl.kernel,
    mesh=_mesh,
    out_type=jax.ShapeDtypeStruct((_N, _D), jnp.float32),
    scratch_types=[
        pltpu.VMEM((_NCH, _C), jnp.int32),
        pltpu.VMEM((3, _C, _D), jnp.float32),
        pltpu.SemaphoreType.DMA,
        pltpu.SemaphoreType.DMA,
        pltpu.SemaphoreType.DMA,
    ],
)
def _gather_rows(idx_hbm, table_hbm, out_hbm, idx_v, rows_v, sem0, sem1, sem2):
    wid = lax.axis_index("s") * _NC + lax.axis_index("c")
    base = wid * _BPW
    pltpu.sync_copy(idx_hbm.at[wid], idx_v)

    sems = (sem0, sem1, sem2)

    def _start(ch, b):
        pltpu.async_copy(table_hbm.at[idx_v.at[ch]], rows_v.at[b], sems[b])

    def _wait(b):
        pltpu.make_async_copy(
            table_hbm.at[idx_v.at[0]], rows_v.at[b], sems[b]
        ).wait()

    def _scatter(ch, b):
        pltpu.sync_copy(rows_v.at[b], out_hbm.at[pl.ds(base + ch * _C, _C)])

    _start(0, 0)
    _start(1, 1)
    _wait(0)
    _start(2, 2)
    _scatter(0, 0)

    def _body(g, carry):
        for k in range(3):
            ch = g * 3 + 1 + k
            b = (1 + k) % 3
            _wait(b)

            @pl.when(ch + 2 < _NCH)
            def _():
                _start(ch + 2, k)

            _scatter(ch, b)
        return carry

    lax.fori_loop(0, (_NCH - 2) // 3, _body, 0)

    _wait(1)
    _scatter(_NCH - 1, 1)


def kernel(inputs, embedding):
    idx = inputs.reshape(_NW, _NCH, _C).astype(jnp.int32)
    out = _gather_rows(idx, embedding)
    return out.reshape(inputs.shape + (_D,))
